# baseline (speedup 1.0000x reference)
import functools
import math

import jax
import jax.numpy as jnp
from jax.experimental import pallas as pl
from jax.experimental.pallas import tpu as pltpu

RB = 512
TT = 512
CACHE = 2

_DOT = dict(preferred_element_type=jnp.float32,
            precision=jax.lax.Precision.HIGHEST)


def _dis_of(deg_raw, diag):
    deg = deg_raw + jnp.where(diag == 0.0, 1.0, 0.0)
    return jnp.where(deg > 0.0,
                     jax.lax.rsqrt(jnp.where(deg > 0.0, deg, 1.0)), 0.0)


def _mega_kernel(x_ref, w1_ref, w2_ref, b1_ref, p1_ref, b2_ref, p2_ref, a_ref,
                 rank1_ref, m1_ref, h2t_ref, s2_ref,
                 deg_ref, diag_ref, xwt_ref, h1t_ref, s1_ref, zw2t_ref,
                 deg2_ref, acache_ref, *, nr, k1):
    ph = pl.program_id(1)
    r = pl.program_id(2)
    rb = a_ref.shape[1]
    n = a_ref.shape[2]
    t = jnp.where(ph % 2 == 0, r, nr - 1 - r)
    sl = pl.ds(t * rb, rb)
    cached = t >= nr - CACHE
    ci = pl.ds(jnp.maximum(t - (nr - CACHE), 0), 1)

    @pl.when(ph == 0)
    def _phase0():
        a = a_ref[0]

        @pl.when(r == 0)
        def _init():
            deg_ref[0] = jnp.zeros_like(deg_ref[0])

        deg_ref[0] += jnp.sum(a, axis=0)

        asq = a_ref[0, :, sl]
        ii = jax.lax.broadcasted_iota(jnp.int32, (rb, rb), 0)
        jj = jax.lax.broadcasted_iota(jnp.int32, (rb, rb), 1)
        diag_ref[0, sl] = jnp.sum(jnp.where(ii == jj, asq, 0.0), axis=1)

        xw = jax.lax.dot_general(x_ref[0], w1_ref[...],
                                 (((1,), (0,)), ((), ())), **_DOT)
        xwt_ref[:, sl] = xw.T

        @pl.when(cached)
        def _stash():
            acache_ref[ci] = a_ref[pl.ds(0, 1)]

    def _ph1_acc(a):
        dis1_blk = _dis_of(deg_ref[0, sl], diag_ref[0, sl])
        ut_blk = dis1_blk[None, :] * xwt_ref[:, sl]

        @pl.when(r == 0)
        def _init():
            h1t_ref[...] = jnp.zeros_like(h1t_ref)

        h1t_ref[...] += jax.lax.dot_general(ut_blk, a,
                                            (((1,), (0,)), ((), ())), **_DOT)

    @pl.when((ph == 1) & ~cached)
    def _phase1_s():
        _ph1_acc(a_ref[0])

    @pl.when((ph == 1) & cached)
    def _phase1_c():
        _ph1_acc(acache_ref[ci][0])

    @pl.when(ph == 1)
    def _phase1():
        @pl.when(r == nr - 1)
        def _fini():
            add1 = jnp.where(diag_ref[0] == 0.0, 1.0, 0.0)
            dis1 = _dis_of(deg_ref[0], diag_ref[0])
            ut_all = dis1[None, :] * xwt_ref[...]
            acc = h1t_ref[...] + add1[None, :] * ut_all
            h1t = jnp.maximum(dis1[None, :] * acc + b1_ref[0][:, None], 0.0)
            h1t_ref[...] = h1t
            p = p1_ref[0]
            pn = jax.lax.rsqrt(jnp.sum(p * p))
            s1_ref[0] = jnp.tanh(jnp.sum(h1t * p[:, None], axis=0) * pn)

    def _ph2_body(a):
        s_all = s1_ref[0]
        s_i = s1_ref[0, sl]
        jidx = jax.lax.broadcasted_iota(jnp.int32, (rb, n), 1)
        iidx = jax.lax.broadcasted_iota(jnp.int32, (rb, n), 0) + t * rb
        gt = (s_all[None, :] > s_i[:, None]).astype(jnp.float32)
        tie = jnp.where((s_all[None, :] == s_i[:, None]) & (jidx < iidx),
                        1.0, 0.0)
        rank = jnp.sum(gt + tie, axis=1)
        m1 = (rank < k1).astype(jnp.float32)
        rank1_ref[0, 0, sl] = rank
        m1_ref[0, 0, sl] = m1
        zt = h1t_ref[:, sl] * (s_i * m1)[None, :]
        zw2t_ref[:, sl] = jax.lax.dot_general(w2_ref[...], zt,
                                              (((0,), (0,)), ((), ())), **_DOT)

        @pl.when(r == 0)
        def _init():
            deg2_ref[0] = jnp.zeros_like(deg2_ref[0])

        deg2_ref[0] += jnp.sum(jnp.where(m1[:, None] > 0.0, a, 0.0), axis=0)

    @pl.when((ph == 2) & ~cached)
    def _phase2_s():
        _ph2_body(a_ref[0])

    @pl.when((ph == 2) & cached)
    def _phase2_c():
        _ph2_body(acache_ref[ci][0])

    def _ph3_acc(a):
        gs_blk = m1_ref[0, 0, sl] * _dis_of(deg2_ref[0, sl], diag_ref[0, sl])
        gt_blk = gs_blk[None, :] * zw2t_ref[:, sl]

        @pl.when(r == 0)
        def _init():
            h2t_ref[0] = jnp.zeros_like(h2t_ref[0])

        h2t_ref[0] += jax.lax.dot_general(gt_blk, a,
                                          (((1,), (0,)), ((), ())), **_DOT)

    @pl.when((ph == 3) & ~cached)
    def _phase3_s():
        _ph3_acc(a_ref[0])

    @pl.when((ph == 3) & cached)
    def _phase3_c():
        _ph3_acc(acache_ref[ci][0])

    @pl.when(ph == 3)
    def _phase3():
        @pl.when(r == nr - 1)
        def _fini():
            add1 = jnp.where(diag_ref[0] == 0.0, 1.0, 0.0)
            dis2 = _dis_of(deg2_ref[0], diag_ref[0])
            gt_all = (m1_ref[0, 0] * dis2)[None, :] * zw2t_ref[...]
            acc = h2t_ref[0] + add1[None, :] * gt_all
            h2t = jnp.maximum(dis2[None, :] * acc + b2_ref[0][:, None], 0.0)
            h2t_ref[0] = h2t
            p = p2_ref[0]
            pn = jax.lax.rsqrt(jnp.sum(p * p))
            s2_ref[0, 0] = jnp.tanh(jnp.sum(h2t * p[:, None], axis=0) * pn)


def _rank2_kernel(s2_ref, rank1_ref, m1_ref, h2t_ref, rank2_ref, vt_ref,
                  *, k2):
    t = pl.program_id(1)
    s_all = s2_ref[0, 0]
    r1_all = rank1_ref[0, 0]
    m_all = m1_ref[0, 0]
    tt = h2t_ref.shape[2]
    sl = pl.ds(t * tt, tt)
    s_i = s2_ref[0, 0, sl]
    r1_i = rank1_ref[0, 0, sl]
    m_i = m1_ref[0, 0, sl]
    gt = (s_all[None, :] > s_i[:, None]).astype(jnp.float32)
    tie = jnp.where((s_all[None, :] == s_i[:, None])
                    & (r1_all[None, :] < r1_i[:, None]), 1.0, 0.0)
    rank2 = jnp.sum(m_all[None, :] * (gt + tie), axis=1)
    valid = m_i * (rank2 < k2).astype(jnp.float32)
    rank2_ref[0, 0, sl] = jnp.where(valid > 0.0, rank2, -1.0)
    vt_ref[0] = h2t_ref[0] * (s_i * valid)[None, :]


def _scatter_kernel(rank2_ref, vt_ref, out_ref, *, k2):
    q = pl.program_id(1)
    tq = out_ref.shape[1]

    @pl.when(q * tq >= k2)
    def _zero():
        out_ref[0] = jnp.zeros_like(out_ref[0])

    @pl.when(q * tq < k2)
    def _dot():
        r2 = rank2_ref[0, 0]
        n = r2.shape[0]
        qidx = (jax.lax.broadcasted_iota(jnp.int32, (n, tq), 1)
                + q * tq).astype(jnp.float32)
        p = jnp.where(r2[:, None] == qidx, 1.0, 0.0)
        outt = jax.lax.dot_general(vt_ref[0], p,
                                   (((1,), (0,)), ((), ())), **_DOT)
        out_ref[0] = outt.T


def _a_stream_index(nr):
    def index_map(b, ph, r):
        t = jnp.where(ph % 2 == 0, r, nr - 1 - r)
        t = jnp.where(ph == 0, t, jnp.minimum(t, nr - CACHE - 1))
        return (b, t, 0)
    return index_map


def kernel(x, A, W1, b1, p1, W2, b2, p2):
    B, N, D0 = x.shape
    D1 = W1.shape[1]
    D2 = W2.shape[1]
    k1 = int(math.ceil(0.8 * N))
    k2 = int(math.ceil(0.5 * k1))
    nr = N // RB
    nt = N // TT
    f32 = jnp.float32
    b1r, p1r = b1.reshape(1, D1), p1.reshape(1, D1)
    b2r, p2r = b2.reshape(1, D2), p2.reshape(1, D2)

    fixed3 = lambda b, ph, r: (b, 0, 0)
    full_n3 = pl.BlockSpec((1, 1, N), fixed3)
    rank1, m1, h2t, s2 = pl.pallas_call(
        functools.partial(_mega_kernel, nr=nr, k1=k1),
        grid=(B, 4, nr),
        in_specs=[
            pl.BlockSpec((1, RB, D0),
                         lambda b, ph, r: (b, jnp.where(ph == 0, r, nr - 1),
                                           0)),
            pl.BlockSpec((D0, D1), lambda b, ph, r: (0, 0)),
            pl.BlockSpec((D1, D2), lambda b, ph, r: (0, 0)),
            pl.BlockSpec((1, D1), lambda b, ph, r: (0, 0)),
            pl.BlockSpec((1, D1), lambda b, ph, r: (0, 0)),
            pl.BlockSpec((1, D2), lambda b, ph, r: (0, 0)),
            pl.BlockSpec((1, D2), lambda b, ph, r: (0, 0)),
            pl.BlockSpec((1, RB, N), _a_stream_index(nr)),
        ],
        out_specs=[full_n3, full_n3,
                   pl.BlockSpec((1, D2, N), fixed3), full_n3],
        out_shape=[jax.ShapeDtypeStruct((B, 1, N), f32),
                   jax.ShapeDtypeStruct((B, 1, N), f32),
                   jax.ShapeDtypeStruct((B, D2, N), f32),
                   jax.ShapeDtypeStruct((B, 1, N), f32)],
        scratch_shapes=[pltpu.VMEM((1, N), f32),
                        pltpu.VMEM((1, N), f32),
                        pltpu.VMEM((D1, N), f32),
                        pltpu.VMEM((D1, N), f32),
                        pltpu.VMEM((1, N), f32),
                        pltpu.VMEM((D2, N), f32),
                        pltpu.VMEM((1, N), f32),
                        pltpu.VMEM((CACHE, RB, N), f32)],
    )(x, W1, W2, b1r, p1r, b2r, p2r, A)

    full_n = pl.BlockSpec((1, 1, N), lambda b, r: (b, 0, 0))
    rank2, vt = pl.pallas_call(
        functools.partial(_rank2_kernel, k2=k2),
        grid=(B, nt),
        in_specs=[full_n, full_n, full_n,
                  pl.BlockSpec((1, D2, TT), lambda b, t: (b, 0, t))],
        out_specs=[full_n,
                   pl.BlockSpec((1, D2, TT), lambda b, t: (b, 0, t))],
        out_shape=[jax.ShapeDtypeStruct((B, 1, N), f32),
                   jax.ShapeDtypeStruct((B, D2, N), f32)],
    )(s2, rank1, m1, h2t)

    out = pl.pallas_call(
        functools.partial(_scatter_kernel, k2=k2),
        grid=(B, nt),
        in_specs=[full_n,
                  pl.BlockSpec((1, D2, N), lambda b, q: (b, 0, 0))],
        out_specs=pl.BlockSpec((1, TT, D2), lambda b, q: (b, q, 0)),
        out_shape=jax.ShapeDtypeStruct((B, N, D2), f32),
    )(rank2, vt)
    return out


# speedup vs baseline: 1.0157x; 1.0157x over previous
import functools
import math

import jax
import jax.numpy as jnp
from jax.experimental import pallas as pl
from jax.experimental.pallas import tpu as pltpu

RB = 512
TT = 512

_DOT = dict(preferred_element_type=jnp.float32,
            precision=jax.lax.Precision.HIGHEST)


def _dis_of(deg_raw, diag):
    deg = deg_raw + jnp.where(diag == 0.0, 1.0, 0.0)
    return jnp.where(deg > 0.0,
                     jax.lax.rsqrt(jnp.where(deg > 0.0, deg, 1.0)), 0.0)


def _mega_kernel(x_ref, w1_ref, w2_ref, b1_ref, p1_ref, b2_ref, p2_ref, a_ref,
                 rank1_ref, m1_ref, h2t_ref, s2_ref,
                 deg_ref, diag_ref, xwt_ref, h1t_ref, s1_ref, zw2t_ref,
                 deg2_ref, *, nr, k1):
    ph = pl.program_id(1)
    r = pl.program_id(2)
    rb = a_ref.shape[1]
    n = a_ref.shape[2]
    t = jnp.where(ph % 2 == 0, r, nr - 1 - r)
    sl = pl.ds(t * rb, rb)

    @pl.when(ph == 0)
    def _phase0():
        a = a_ref[0]

        @pl.when(r == 0)
        def _init():
            deg_ref[0] = jnp.zeros_like(deg_ref[0])

        deg_ref[0] += jnp.sum(a, axis=0)

        asq = a_ref[0, :, sl]
        ii = jax.lax.broadcasted_iota(jnp.int32, (rb, rb), 0)
        jj = jax.lax.broadcasted_iota(jnp.int32, (rb, rb), 1)
        diag_ref[0, sl] = jnp.sum(jnp.where(ii == jj, asq, 0.0), axis=1)

        xw = jax.lax.dot_general(x_ref[0], w1_ref[...],
                                 (((1,), (0,)), ((), ())), **_DOT)
        xwt_ref[:, sl] = xw.T

    def _ph1_acc(a):
        dis1_blk = _dis_of(deg_ref[0, sl], diag_ref[0, sl])
        ut_blk = dis1_blk[None, :] * xwt_ref[:, sl]

        @pl.when(r == 0)
        def _init():
            h1t_ref[...] = jnp.zeros_like(h1t_ref)

        h1t_ref[...] += jax.lax.dot_general(ut_blk, a,
                                            (((1,), (0,)), ((), ())), **_DOT)

    @pl.when(ph == 1)
    def _phase1():
        _ph1_acc(a_ref[0])

        @pl.when(r == nr - 1)
        def _fini():
            add1 = jnp.where(diag_ref[0] == 0.0, 1.0, 0.0)
            dis1 = _dis_of(deg_ref[0], diag_ref[0])
            ut_all = dis1[None, :] * xwt_ref[...]
            acc = h1t_ref[...] + add1[None, :] * ut_all
            h1t = jnp.maximum(dis1[None, :] * acc + b1_ref[0][:, None], 0.0)
            h1t_ref[...] = h1t
            p = p1_ref[0]
            pn = jax.lax.rsqrt(jnp.sum(p * p))
            s1_ref[0] = jnp.tanh(jnp.sum(h1t * p[:, None], axis=0) * pn)

    def _ph2_body(a):
        s_all = s1_ref[0]
        s_i = s1_ref[0, sl]
        jidx = jax.lax.broadcasted_iota(jnp.int32, (rb, n), 1)
        iidx = jax.lax.broadcasted_iota(jnp.int32, (rb, n), 0) + t * rb
        gt = (s_all[None, :] > s_i[:, None]).astype(jnp.float32)
        tie = jnp.where((s_all[None, :] == s_i[:, None]) & (jidx < iidx),
                        1.0, 0.0)
        rank = jnp.sum(gt + tie, axis=1)
        m1 = (rank < k1).astype(jnp.float32)
        rank1_ref[0, 0, sl] = rank
        m1_ref[0, 0, sl] = m1
        zt = h1t_ref[:, sl] * (s_i * m1)[None, :]
        zw2t_ref[:, sl] = jax.lax.dot_general(w2_ref[...], zt,
                                              (((0,), (0,)), ((), ())), **_DOT)

        @pl.when(r == 0)
        def _init():
            deg2_ref[0] = jnp.zeros_like(deg2_ref[0])

        deg2_ref[0] += jnp.sum(jnp.where(m1[:, None] > 0.0, a, 0.0), axis=0)

    @pl.when(ph == 2)
    def _phase2():
        _ph2_body(a_ref[0])

    def _ph3_acc(a):
        gs_blk = m1_ref[0, 0, sl] * _dis_of(deg2_ref[0, sl], diag_ref[0, sl])
        gt_blk = gs_blk[None, :] * zw2t_ref[:, sl]

        @pl.when(r == 0)
        def _init():
            h2t_ref[0] = jnp.zeros_like(h2t_ref[0])

        h2t_ref[0] += jax.lax.dot_general(gt_blk, a,
                                          (((1,), (0,)), ((), ())), **_DOT)

    @pl.when(ph == 3)
    def _phase3():
        _ph3_acc(a_ref[0])

        @pl.when(r == nr - 1)
        def _fini():
            add1 = jnp.where(diag_ref[0] == 0.0, 1.0, 0.0)
            dis2 = _dis_of(deg2_ref[0], diag_ref[0])
            gt_all = (m1_ref[0, 0] * dis2)[None, :] * zw2t_ref[...]
            acc = h2t_ref[0] + add1[None, :] * gt_all
            h2t = jnp.maximum(dis2[None, :] * acc + b2_ref[0][:, None], 0.0)
            h2t_ref[0] = h2t
            p = p2_ref[0]
            pn = jax.lax.rsqrt(jnp.sum(p * p))
            s2_ref[0, 0] = jnp.tanh(jnp.sum(h2t * p[:, None], axis=0) * pn)


def _rank2_kernel(s2_ref, rank1_ref, m1_ref, h2t_ref, rank2_ref, vt_ref,
                  *, k2):
    t = pl.program_id(1)
    s_all = s2_ref[0, 0]
    r1_all = rank1_ref[0, 0]
    m_all = m1_ref[0, 0]
    tt = h2t_ref.shape[2]
    sl = pl.ds(t * tt, tt)
    s_i = s2_ref[0, 0, sl]
    r1_i = rank1_ref[0, 0, sl]
    m_i = m1_ref[0, 0, sl]
    gt = (s_all[None, :] > s_i[:, None]).astype(jnp.float32)
    tie = jnp.where((s_all[None, :] == s_i[:, None])
                    & (r1_all[None, :] < r1_i[:, None]), 1.0, 0.0)
    rank2 = jnp.sum(m_all[None, :] * (gt + tie), axis=1)
    valid = m_i * (rank2 < k2).astype(jnp.float32)
    rank2_ref[0, 0, sl] = jnp.where(valid > 0.0, rank2, -1.0)
    vt_ref[0] = h2t_ref[0] * (s_i * valid)[None, :]


def _scatter_kernel(rank2_ref, vt_ref, out_ref, *, k2):
    q = pl.program_id(1)
    tq = out_ref.shape[1]

    @pl.when(q * tq >= k2)
    def _zero():
        out_ref[0] = jnp.zeros_like(out_ref[0])

    @pl.when(q * tq < k2)
    def _dot():
        r2 = rank2_ref[0, 0]
        n = r2.shape[0]
        qidx = (jax.lax.broadcasted_iota(jnp.int32, (n, tq), 1)
                + q * tq).astype(jnp.float32)
        p = jnp.where(r2[:, None] == qidx, 1.0, 0.0)
        outt = jax.lax.dot_general(vt_ref[0], p,
                                   (((1,), (0,)), ((), ())), **_DOT)
        out_ref[0] = outt.T


def _a_stream_index(nr):
    def index_map(b, ph, r):
        t = jnp.where(ph % 2 == 0, r, nr - 1 - r)
        return (b, t, 0)
    return index_map


def kernel(x, A, W1, b1, p1, W2, b2, p2):
    B, N, D0 = x.shape
    D1 = W1.shape[1]
    D2 = W2.shape[1]
    k1 = int(math.ceil(0.8 * N))
    k2 = int(math.ceil(0.5 * k1))
    nr = N // RB
    nt = N // TT
    f32 = jnp.float32
    b1r, p1r = b1.reshape(1, D1), p1.reshape(1, D1)
    b2r, p2r = b2.reshape(1, D2), p2.reshape(1, D2)

    fixed3 = lambda b, ph, r: (b, 0, 0)
    full_n3 = pl.BlockSpec((1, 1, N), fixed3)
    rank1, m1, h2t, s2 = pl.pallas_call(
        functools.partial(_mega_kernel, nr=nr, k1=k1),
        grid=(B, 4, nr),
        in_specs=[
            pl.BlockSpec((1, RB, D0),
                         lambda b, ph, r: (b, jnp.where(ph == 0, r, nr - 1),
                                           0)),
            pl.BlockSpec((D0, D1), lambda b, ph, r: (0, 0)),
            pl.BlockSpec((D1, D2), lambda b, ph, r: (0, 0)),
            pl.BlockSpec((1, D1), lambda b, ph, r: (0, 0)),
            pl.BlockSpec((1, D1), lambda b, ph, r: (0, 0)),
            pl.BlockSpec((1, D2), lambda b, ph, r: (0, 0)),
            pl.BlockSpec((1, D2), lambda b, ph, r: (0, 0)),
            pl.BlockSpec((1, RB, N), _a_stream_index(nr)),
        ],
        out_specs=[full_n3, full_n3,
                   pl.BlockSpec((1, D2, N), fixed3), full_n3],
        out_shape=[jax.ShapeDtypeStruct((B, 1, N), f32),
                   jax.ShapeDtypeStruct((B, 1, N), f32),
                   jax.ShapeDtypeStruct((B, D2, N), f32),
                   jax.ShapeDtypeStruct((B, 1, N), f32)],
        scratch_shapes=[pltpu.VMEM((1, N), f32),
                        pltpu.VMEM((1, N), f32),
                        pltpu.VMEM((D1, N), f32),
                        pltpu.VMEM((D1, N), f32),
                        pltpu.VMEM((1, N), f32),
                        pltpu.VMEM((D2, N), f32),
                        pltpu.VMEM((1, N), f32)],
    )(x, W1, W2, b1r, p1r, b2r, p2r, A)

    full_n = pl.BlockSpec((1, 1, N), lambda b, r: (b, 0, 0))
    rank2, vt = pl.pallas_call(
        functools.partial(_rank2_kernel, k2=k2),
        grid=(B, nt),
        in_specs=[full_n, full_n, full_n,
                  pl.BlockSpec((1, D2, TT), lambda b, t: (b, 0, t))],
        out_specs=[full_n,
                   pl.BlockSpec((1, D2, TT), lambda b, t: (b, 0, t))],
        out_shape=[jax.ShapeDtypeStruct((B, 1, N), f32),
                   jax.ShapeDtypeStruct((B, D2, N), f32)],
    )(s2, rank1, m1, h2t)

    out = pl.pallas_call(
        functools.partial(_scatter_kernel, k2=k2),
        grid=(B, nt),
        in_specs=[full_n,
                  pl.BlockSpec((1, D2, N), lambda b, q: (b, 0, 0))],
        out_specs=pl.BlockSpec((1, TT, D2), lambda b, q: (b, q, 0)),
        out_shape=jax.ShapeDtypeStruct((B, N, D2), f32),
    )(rank2, vt)
    return out


# speedup vs baseline: 1.0961x; 1.0792x over previous
import functools
import math

import jax
import jax.numpy as jnp
from jax.experimental import pallas as pl
from jax.experimental.pallas import tpu as pltpu

RB = 512
TT = 512

_DOT = dict(preferred_element_type=jnp.float32,
            precision=jax.lax.Precision.HIGHEST)


def _dis_of(deg_raw, diag):
    deg = deg_raw + jnp.where(diag == 0.0, 1.0, 0.0)
    return jnp.where(deg > 0.0,
                     jax.lax.rsqrt(jnp.where(deg > 0.0, deg, 1.0)), 0.0)


def _mega_kernel(x_ref, w1_ref, w2_ref, b1_ref, p1_ref, b2_ref, p2_ref, a_ref,
                 rank1_ref, m1_ref, h2t_ref, s2_ref,
                 deg_ref, diag_ref, xwt_ref, h1t_ref, s1_ref, zw2t_ref,
                 deg2_ref, *, nr, k1):
    ph = pl.program_id(1)
    r = pl.program_id(2)
    rb = a_ref.shape[1]
    n = a_ref.shape[2]
    t = jnp.where(ph % 2 == 0, r, nr - 1 - r)
    sl = pl.ds(t * rb, rb)

    @pl.when(ph == 0)
    def _phase0():
        a = a_ref[0]

        @pl.when(r == 0)
        def _init():
            deg_ref[0] = jnp.zeros_like(deg_ref[0])

        deg_ref[0] += jnp.sum(a, axis=0)

        asq = a_ref[0, :, sl]
        ii = jax.lax.broadcasted_iota(jnp.int32, (rb, rb), 0)
        jj = jax.lax.broadcasted_iota(jnp.int32, (rb, rb), 1)
        diag_ref[0, sl] = jnp.sum(jnp.where(ii == jj, asq, 0.0), axis=1)

        xw = jax.lax.dot_general(x_ref[0], w1_ref[...],
                                 (((1,), (0,)), ((), ())), **_DOT)
        xwt_ref[:, sl] = xw.T

    @pl.when(ph == 1)
    def _phase1():
        dis1_blk = _dis_of(deg_ref[0, sl], diag_ref[0, sl])
        ut_blk = dis1_blk[None, :] * xwt_ref[:, sl]

        @pl.when(r == 0)
        def _init():
            h1t_ref[...] = jnp.zeros_like(h1t_ref)

        h1t_ref[...] += jax.lax.dot_general(ut_blk, a_ref[0],
                                            (((1,), (0,)), ((), ())), **_DOT)

        @pl.when(r == nr - 1)
        def _fini():
            add1 = jnp.where(diag_ref[0] == 0.0, 1.0, 0.0)
            dis1 = _dis_of(deg_ref[0], diag_ref[0])
            ut_all = dis1[None, :] * xwt_ref[...]
            acc = h1t_ref[...] + add1[None, :] * ut_all
            h1t = jnp.maximum(dis1[None, :] * acc + b1_ref[0][:, None], 0.0)
            h1t_ref[...] = h1t
            p = p1_ref[0]
            pn = jax.lax.rsqrt(jnp.sum(p * p))
            s1_ref[0] = jnp.tanh(jnp.sum(h1t * p[:, None], axis=0) * pn)

    @pl.when(ph == 2)
    def _phase2():
        s_all = s1_ref[0]
        s_i = s1_ref[0, sl]
        jidx = jax.lax.broadcasted_iota(jnp.int32, (rb, n), 1)
        iidx = jax.lax.broadcasted_iota(jnp.int32, (rb, n), 0) + t * rb
        gt = (s_all[None, :] > s_i[:, None]).astype(jnp.float32)
        tie = jnp.where((s_all[None, :] == s_i[:, None]) & (jidx < iidx),
                        1.0, 0.0)
        rank = jnp.sum(gt + tie, axis=1)
        m1 = (rank < k1).astype(jnp.float32)
        rank1_ref[0, 0, sl] = rank
        m1_ref[0, 0, sl] = m1
        zt = h1t_ref[:, sl] * (s_i * m1)[None, :]
        zw2t_ref[:, sl] = jax.lax.dot_general(w2_ref[...], zt,
                                              (((0,), (0,)), ((), ())), **_DOT)

        @pl.when(r == 0)
        def _init():
            deg2_ref[0] = jnp.zeros_like(deg2_ref[0])

        deg2_ref[0] += jnp.sum(jnp.where(m1[:, None] > 0.0, a_ref[0], 0.0),
                               axis=0)

    @pl.when(ph == 3)
    def _phase3():
        gs_blk = m1_ref[0, 0, sl] * _dis_of(deg2_ref[0, sl], diag_ref[0, sl])
        gt_blk = gs_blk[None, :] * zw2t_ref[:, sl]

        @pl.when(r == 0)
        def _init():
            h2t_ref[0] = jnp.zeros_like(h2t_ref[0])

        h2t_ref[0] += jax.lax.dot_general(gt_blk, a_ref[0],
                                          (((1,), (0,)), ((), ())), **_DOT)

        @pl.when(r == nr - 1)
        def _fini():
            add1 = jnp.where(diag_ref[0] == 0.0, 1.0, 0.0)
            dis2 = _dis_of(deg2_ref[0], diag_ref[0])
            gt_all = (m1_ref[0, 0] * dis2)[None, :] * zw2t_ref[...]
            acc = h2t_ref[0] + add1[None, :] * gt_all
            h2t = jnp.maximum(dis2[None, :] * acc + b2_ref[0][:, None], 0.0)
            h2t_ref[0] = h2t
            p = p2_ref[0]
            pn = jax.lax.rsqrt(jnp.sum(p * p))
            s2_ref[0, 0] = jnp.tanh(jnp.sum(h2t * p[:, None], axis=0) * pn)


def _rank2_kernel(s2_ref, rank1_ref, m1_ref, h2t_ref, rank2_ref, vt_ref,
                  *, k2):
    t = pl.program_id(1)
    s_all = s2_ref[0, 0]
    r1_all = rank1_ref[0, 0]
    m_all = m1_ref[0, 0]
    tt = h2t_ref.shape[2]
    sl = pl.ds(t * tt, tt)
    s_i = s2_ref[0, 0, sl]
    r1_i = rank1_ref[0, 0, sl]
    m_i = m1_ref[0, 0, sl]
    gt = (s_all[None, :] > s_i[:, None]).astype(jnp.float32)
    tie = jnp.where((s_all[None, :] == s_i[:, None])
                    & (r1_all[None, :] < r1_i[:, None]), 1.0, 0.0)
    rank2 = jnp.sum(m_all[None, :] * (gt + tie), axis=1)
    valid = m_i * (rank2 < k2).astype(jnp.float32)
    rank2_ref[0, 0, sl] = jnp.where(valid > 0.0, rank2, -1.0)
    vt_ref[0] = h2t_ref[0] * (s_i * valid)[None, :]


def _scatter_kernel(rank2_ref, vt_ref, out_ref, *, k2):
    q = pl.program_id(1)
    tq = out_ref.shape[1]

    @pl.when(q * tq >= k2)
    def _zero():
        out_ref[0] = jnp.zeros_like(out_ref[0])

    @pl.when(q * tq < k2)
    def _dot():
        r2 = rank2_ref[0, 0]
        n = r2.shape[0]
        qidx = (jax.lax.broadcasted_iota(jnp.int32, (n, tq), 1)
                + q * tq).astype(jnp.float32)
        p = jnp.where(r2[:, None] == qidx, 1.0, 0.0)
        outt = jax.lax.dot_general(vt_ref[0], p,
                                   (((1,), (0,)), ((), ())), **_DOT)
        out_ref[0] = outt.T


def _a_stream_index(nr):
    def index_map(b, ph, r):
        t = jnp.where(ph % 2 == 0, r, nr - 1 - r)
        return (b, t, 0)
    return index_map


def kernel(x, A, W1, b1, p1, W2, b2, p2):
    B, N, D0 = x.shape
    D1 = W1.shape[1]
    D2 = W2.shape[1]
    k1 = int(math.ceil(0.8 * N))
    k2 = int(math.ceil(0.5 * k1))
    nr = N // RB
    nt = N // TT
    f32 = jnp.float32
    b1r, p1r = b1.reshape(1, D1), p1.reshape(1, D1)
    b2r, p2r = b2.reshape(1, D2), p2.reshape(1, D2)

    fixed3 = lambda b, ph, r: (b, 0, 0)
    full_n3 = pl.BlockSpec((1, 1, N), fixed3)
    rank1, m1, h2t, s2 = pl.pallas_call(
        functools.partial(_mega_kernel, nr=nr, k1=k1),
        grid=(B, 4, nr),
        in_specs=[
            pl.BlockSpec((1, RB, D0),
                         lambda b, ph, r: (b, jnp.where(ph == 0, r, nr - 1),
                                           0)),
            pl.BlockSpec((D0, D1), lambda b, ph, r: (0, 0)),
            pl.BlockSpec((D1, D2), lambda b, ph, r: (0, 0)),
            pl.BlockSpec((1, D1), lambda b, ph, r: (0, 0)),
            pl.BlockSpec((1, D1), lambda b, ph, r: (0, 0)),
            pl.BlockSpec((1, D2), lambda b, ph, r: (0, 0)),
            pl.BlockSpec((1, D2), lambda b, ph, r: (0, 0)),
            pl.BlockSpec((1, RB, N), _a_stream_index(nr)),
        ],
        out_specs=[full_n3, full_n3,
                   pl.BlockSpec((1, D2, N), fixed3), full_n3],
        out_shape=[jax.ShapeDtypeStruct((B, 1, N), f32),
                   jax.ShapeDtypeStruct((B, 1, N), f32),
                   jax.ShapeDtypeStruct((B, D2, N), f32),
                   jax.ShapeDtypeStruct((B, 1, N), f32)],
        scratch_shapes=[pltpu.VMEM((1, N), f32),
                        pltpu.VMEM((1, N), f32),
                        pltpu.VMEM((D1, N), f32),
                        pltpu.VMEM((D1, N), f32),
                        pltpu.VMEM((1, N), f32),
                        pltpu.VMEM((D2, N), f32),
                        pltpu.VMEM((1, N), f32)],
    )(x, W1, W2, b1r, p1r, b2r, p2r, A)

    full_n = pl.BlockSpec((1, 1, N), lambda b, r: (b, 0, 0))
    rank2, vt = pl.pallas_call(
        functools.partial(_rank2_kernel, k2=k2),
        grid=(B, nt),
        in_specs=[full_n, full_n, full_n,
                  pl.BlockSpec((1, D2, TT), lambda b, t: (b, 0, t))],
        out_specs=[full_n,
                   pl.BlockSpec((1, D2, TT), lambda b, t: (b, 0, t))],
        out_shape=[jax.ShapeDtypeStruct((B, 1, N), f32),
                   jax.ShapeDtypeStruct((B, D2, N), f32)],
    )(s2, rank1, m1, h2t)

    out = pl.pallas_call(
        functools.partial(_scatter_kernel, k2=k2),
        grid=(B, nt),
        in_specs=[full_n,
                  pl.BlockSpec((1, D2, N), lambda b, q: (b, 0, 0))],
        out_specs=pl.BlockSpec((1, TT, D2), lambda b, q: (b, q, 0)),
        out_shape=jax.ShapeDtypeStruct((B, N, D2), f32),
    )(rank2, vt)
    return out
